# baseline (speedup 1.0000x reference)
import jax
import jax.numpy as jnp
from jax.experimental import pallas as pl
from jax.experimental.pallas import tpu as pltpu

_M = 100000
_D = 64
_B = 1024
_TM = 512


def _fused_kernel(m_ref, idxc_ref, idxr_ref, x_ref, feat_ref,
                  out_ref, newf_ref, xn_ref, valid_ref):
    i = pl.program_id(0)

    @pl.when(i == 0)
    def _prologue():
        x = x_ref[...]
        xn_ref[...] = x / (jnp.sqrt(jnp.sum(x * x, axis=1, keepdims=True)) + 1e-12)
        eq = idxc_ref[...] == idxr_ref[...]
        ii = jax.lax.broadcasted_iota(jnp.int32, (_B, _B), 0)
        jj = jax.lax.broadcasted_iota(jnp.int32, (_B, _B), 1)
        dup_later = jnp.any(eq & (jj > ii), axis=1, keepdims=True)
        valid_ref[...] = jnp.where(dup_later, 0.0, 1.0)

    base = i * _TM
    feat = feat_ref[...]
    row_ok = (base + jax.lax.broadcasted_iota(jnp.int32, (_TM, 1), 0)) < _M
    featm = jnp.where(row_ok, feat, 0.0)
    xn = xn_ref[...]

    out_ref[...] = jax.lax.dot_general(
        xn, featm, (((1,), (1,)), ((), ())), preferred_element_type=jnp.float32)

    cols = base + jax.lax.broadcasted_iota(jnp.int32, (_B, _TM), 1)
    hit = (idxc_ref[...] == cols).astype(jnp.float32) * valid_ref[...]

    g = jax.lax.dot_general(
        hit, featm, (((1,), (0,)), ((), ())), preferred_element_type=jnp.float32)
    m = m_ref[0, 0]
    upd = m * g + (1.0 - m) * xn
    upd = upd / (jnp.sqrt(jnp.sum(upd * upd, axis=1, keepdims=True)) + 1e-12)

    up_rows = jax.lax.dot_general(
        hit, upd, (((0,), (0,)), ((), ())), preferred_element_type=jnp.float32)
    hit_any = jnp.sum(hit, axis=0)[:, None]
    newf_ref[...] = jnp.where(hit_any > 0.0, up_rows, feat)


def kernel(inputs, indexes, features, momentum):
    m2 = jnp.asarray(momentum, jnp.float32).reshape(1, 1)
    idx_col = indexes.reshape(_B, 1)
    idx_row = indexes.reshape(1, _B)
    grid = pl.cdiv(_M, _TM)
    out, newf = pl.pallas_call(
        _fused_kernel,
        grid=(grid,),
        in_specs=[
            pl.BlockSpec(memory_space=pltpu.SMEM),
            pl.BlockSpec((_B, 1), lambda i: (0, 0)),
            pl.BlockSpec((1, _B), lambda i: (0, 0)),
            pl.BlockSpec((_B, _D), lambda i: (0, 0)),
            pl.BlockSpec((_TM, _D), lambda i: (i, 0)),
        ],
        out_specs=[
            pl.BlockSpec((_B, _TM), lambda i: (0, i)),
            pl.BlockSpec((_TM, _D), lambda i: (i, 0)),
        ],
        out_shape=[
            jax.ShapeDtypeStruct((_B, _M), jnp.float32),
            jax.ShapeDtypeStruct((_M, _D), jnp.float32),
        ],
        scratch_shapes=[
            pltpu.VMEM((_B, _D), jnp.float32),
            pltpu.VMEM((_B, 1), jnp.float32),
        ],
    )(m2, idx_col, idx_row, inputs, features)
    return out, newf
